# baseline (speedup 1.0000x reference)
import functools

import jax
import jax.numpy as jnp
from jax import lax
from jax.experimental import pallas as pl
from jax.experimental.pallas import tpu as pltpu
from jax.experimental.pallas import tpu_sc as plsc

_N = 10000
_E = 320000
_D = 128
_H = 128
_M = 3
_DEPTH = 3
_H3 = _H // 3
_H9 = _H3 // 3

_NC = 2
_NS = 16
_NW = _NC * _NS
_EPT = _E // _NW
_K = 80
_EPTP = 10240
_NCHUNK = _EPTP // _K
_NPAD = 10240
_RPT = _NPAD // _NS

@functools.partial(
    pl.kernel,
    out_type=jax.ShapeDtypeStruct((_M * 2 * _NPAD, _H), jnp.float32),
    mesh=plsc.VectorSubcoreMesh(core_axis_name="c", subcore_axis_name="s"),
    scratch_types=[
        pltpu.VMEM((4, _K), jnp.int32),
        pltpu.VMEM((_NCHUNK, _K), jnp.int32),
        pltpu.VMEM((2, _K, _H), jnp.float32),
        pltpu.VMEM_SHARED((_NPAD, _H), jnp.float32),
        ((pltpu.SemaphoreType.DMA, pltpu.SemaphoreType.DMA),
         (pltpu.SemaphoreType.DMA, pltpu.SemaphoreType.DMA,
          pltpu.SemaphoreType.DMA, pltpu.SemaphoreType.DMA)),
    ],
)
def _sc_segsum(h_hbm, srcf_hbm, dst_hbm, zeros_hbm, agg_hbm,
               srcr, dst_v, rows_v, acc, sems):
    gsem, isem = sems
    c = lax.axis_index("c")
    s = lax.axis_index("s")
    wid = c * _NS + s
    pltpu.sync_copy(dst_hbm.at[wid], dst_v)

    def ifetch(base_s, q, slot):
        pltpu.async_copy(srcf_hbm.at[base_s + q], srcr.at[slot], isem[slot])

    def iwait(slot):
        pltpu.make_async_copy(srcf_hbm.at[0], srcr.at[slot],
                              isem[slot]).wait()

    def gather(slot, b):
        return pltpu.async_copy(h_hbm.at[srcr.at[slot]], rows_v.at[b],
                                gsem[b])

    def scat(b, q):
        pltpu.sync_copy(rows_v.at[b], acc.at[dst_v.at[q]], add=True)

    for m in range(_M):
        base_s = (m * _NW + wid) * _NCHUNK
        ifetch(base_s, 0, 0)
        ifetch(base_s, 1, 1)
        pltpu.sync_copy(zeros_hbm, acc.at[pl.ds(s * _RPT, _RPT)])
        plsc.subcore_barrier()

        def quad(j4, carry):
            q = 4 * j4
            ifetch(base_s, q + 2, 2)
            ifetch(base_s, q + 3, 3)
            iwait(0)
            d0 = gather(0, 0)
            iwait(1)
            d1 = gather(1, 1)
            d0.wait()
            scat(0, q)
            d1.wait()
            scat(1, q + 1)
            ifetch(base_s, q + 4, 0)
            ifetch(base_s, q + 5, 1)
            iwait(2)
            d2 = gather(2, 0)
            iwait(3)
            d3 = gather(3, 1)
            d2.wait()
            scat(0, q + 2)
            d3.wait()
            scat(1, q + 3)
            return carry

        lax.fori_loop(0, _NCHUNK // 4 - 1, quad, 0)
        qe = _NCHUNK - 4
        ifetch(base_s, qe + 2, 2)
        ifetch(base_s, qe + 3, 3)
        iwait(0)
        d0 = gather(0, 0)
        iwait(1)
        d1 = gather(1, 1)
        d0.wait()
        scat(0, qe)
        d1.wait()
        scat(1, qe + 1)
        iwait(2)
        d2 = gather(2, 0)
        iwait(3)
        d3 = gather(3, 1)
        d2.wait()
        scat(0, qe + 2)
        d3.wait()
        scat(1, qe + 3)

        plsc.subcore_barrier()
        row0 = (2 * m + c) * _NPAD + s * _RPT
        pltpu.sync_copy(acc.at[pl.ds(s * _RPT, _RPT)],
                        agg_hbm.at[pl.ds(row0, _RPT)])


_BN = 1000
_NB = _N // _BN


def _h0_body(x_ref, wi_ref, out_ref):
    x = x_ref[...]
    for m in range(_M):
        out_ref[m] = jnp.maximum(lax.dot(x, wi_ref[m]), 0.0)


_h0_call = pl.pallas_call(
    _h0_body,
    grid=(_NB,),
    in_specs=[
        pl.BlockSpec((_BN, _D), lambda i: (i, 0)),
        pl.BlockSpec((_M, _D, _H), lambda i: (0, 0, 0)),
    ],
    out_specs=pl.BlockSpec((_M, _BN, _H), lambda i: (0, i, 0)),
    out_shape=jax.ShapeDtypeStruct((_M, _N, _H), jnp.float32),
)


def _upd_body(h0_ref, agg_ref, wh_ref, out_ref):
    for m in range(_M):
        a = agg_ref[m, 0] + agg_ref[m, 1]
        out_ref[m] = jnp.maximum(
            h0_ref[m] + lax.dot(a, wh_ref[m]), 0.0)


_upd_call = pl.pallas_call(
    _upd_body,
    grid=(_NB,),
    in_specs=[
        pl.BlockSpec((_M, _BN, _H), lambda i: (0, i, 0)),
        pl.BlockSpec((_M, 2, _BN, _H), lambda i: (0, 0, i, 0)),
        pl.BlockSpec((_M, _H, _H), lambda i: (0, 0, 0)),
    ],
    out_specs=pl.BlockSpec((_M, _BN, _H), lambda i: (0, i, 0)),
    out_shape=jax.ShapeDtypeStruct((_M, _N, _H), jnp.float32),
)


def _readout_body(x_ref, h_ref, wo_ref, bo_ref, w1_ref, b1_ref, w2_ref,
                  b2_ref, cw1_ref, cb1_ref, cw2_ref, cb2_ref, cw3_ref,
                  cb3_ref, out_ref, acc_ref):
    i = pl.program_id(0)

    @pl.when(i == 0)
    def _():
        acc_ref[...] = jnp.zeros_like(acc_ref)

    x = x_ref[...]
    for m in range(_M):
        ah = jnp.maximum(
            lax.dot(x, wo_ref[m, :_D, :])
            + lax.dot(h_ref[m], wo_ref[m, _D:, :])
            + bo_ref[m][None, :], 0.0)
        acc_ref[m, :] = acc_ref[m, :] + jnp.sum(ah, axis=0)

    @pl.when(i == _NB - 1)
    def _():
        def rb(v):
            return v.astype(jnp.bfloat16).astype(jnp.float32)

        total = 0.0
        for m in range(_M):
            e = rb(acc_ref[m, :] * (1.0 / _N))
            t = jnp.maximum(
                jnp.sum(e[:, None] * rb(w1_ref[m]), axis=0) + b1_ref[m], 0.0)
            temp = jnp.sum(rb(t) * rb(w2_ref[m])) + b2_ref[m]
            z = jnp.maximum(
                jnp.sum(e[:, None] * rb(cw1_ref[m]), axis=0) + cb1_ref[m], 0.0)
            z2 = jnp.maximum(
                jnp.sum(rb(z)[:, None] * rb(cw2_ref[m]), axis=0) + cb2_ref[m], 0.0)
            coef = jnp.sum(rb(z2) * rb(cw3_ref[m])) + cb3_ref[m]
            total = total + temp * coef
        out_ref[...] = jnp.reshape(total, (1, 1))


_readout_call = pl.pallas_call(
    _readout_body,
    grid=(_NB,),
    in_specs=[
        pl.BlockSpec((_BN, _D), lambda i: (i, 0)),
        pl.BlockSpec((_M, _BN, _H), lambda i: (0, i, 0)),
        pl.BlockSpec((_M, _D + _H, _H), lambda i: (0, 0, 0)),
        pl.BlockSpec((_M, _H), lambda i: (0, 0)),
        pl.BlockSpec((_M, _H, _H), lambda i: (0, 0, 0)),
        pl.BlockSpec((_M, _H), lambda i: (0, 0)),
        pl.BlockSpec((_M, _H), lambda i: (0, 0)),
        pl.BlockSpec((_M,), lambda i: (0,)),
        pl.BlockSpec((_M, _H, _H3), lambda i: (0, 0, 0)),
        pl.BlockSpec((_M, _H3), lambda i: (0, 0)),
        pl.BlockSpec((_M, _H3, _H9), lambda i: (0, 0, 0)),
        pl.BlockSpec((_M, _H9), lambda i: (0, 0)),
        pl.BlockSpec((_M, _H9), lambda i: (0, 0)),
        pl.BlockSpec((_M,), lambda i: (0,)),
    ],
    out_specs=pl.BlockSpec((1, 1), lambda i: (0, 0)),
    out_shape=jax.ShapeDtypeStruct((1, 1), jnp.float32),
    scratch_shapes=[pltpu.VMEM((_M, _H), jnp.float32)],
)


def kernel(x, edge_index, W_i, W_h, W_o, b_o, ffn_W1, ffn_b1, ffn_W2, ffn_b2,
           c_W1, c_b1, c_W2, c_b2, c_W3, c_b3):
    src = edge_index[0]
    dst = edge_index[1]
    npad_e = _EPTP - _EPT
    src_r = jnp.concatenate(
        [src.reshape(_NW, _EPT),
         jnp.zeros((_NW, npad_e), jnp.int32)], axis=1)
    pad_rows = (_N + (jnp.arange(_NW, dtype=jnp.int32)[:, None] * 7
                      + jnp.arange(npad_e, dtype=jnp.int32)[None, :])
                % (_NPAD - _N))
    dst_r = jnp.concatenate(
        [dst.reshape(_NW, _EPT), pad_rows], axis=1)
    offs = (jnp.arange(_M, dtype=jnp.int32) * _N)[:, None, None]
    src_m = (src_r[None] + offs).reshape(_M * _NW * _NCHUNK, _K)
    dst_r = dst_r.reshape(_NW, _NCHUNK, _K)
    zeros = jnp.zeros((_RPT, _H), jnp.float32)

    h0 = _h0_call(x, W_i)
    h = h0
    for _ in range(_DEPTH):
        agg_flat = _sc_segsum(h.reshape(_M * _N, _H), src_m, dst_r, zeros)
        agg = agg_flat.reshape(_M, 2, _NPAD, _H)
        h = _upd_call(h0, agg, W_h)
    out = _readout_call(x, h, W_o, b_o, ffn_W1, ffn_b1, ffn_W2[..., 0],
                        ffn_b2[..., 0], c_W1, c_b1, c_W2, c_b2, c_W3[..., 0],
                        c_b3[..., 0])
    return out


# speedup vs baseline: 1.9472x; 1.9472x over previous
import functools

import jax
import jax.numpy as jnp
from jax import lax
from jax.experimental import pallas as pl
from jax.experimental.pallas import tpu as pltpu
from jax.experimental.pallas import tpu_sc as plsc

_N = 10000
_E = 320000
_D = 128
_H = 128
_M = 3
_DEPTH = 3
_H3 = _H // 3
_H9 = _H3 // 3

_NC = 2
_NS = 16
_NW = _NC * _NS
_EPT = _E // _NW
_K = 120
_EPTP = 10080
_NCHUNK = _EPTP // _K
_NPAD = 10240
_RPT = _NPAD // _NS

@functools.partial(
    pl.kernel,
    out_type=jax.ShapeDtypeStruct((_M * 2 * _NPAD, _H), jnp.float32),
    mesh=plsc.VectorSubcoreMesh(core_axis_name="c", subcore_axis_name="s"),
    scratch_types=[
        pltpu.VMEM((_NCHUNK, _K), jnp.int32),
        pltpu.VMEM((4, _K), jnp.int32),
        pltpu.VMEM((2, _K, _H), jnp.float32),
        pltpu.VMEM_SHARED((_NPAD, _H), jnp.float32),
        ((pltpu.SemaphoreType.DMA, pltpu.SemaphoreType.DMA),
         (pltpu.SemaphoreType.DMA, pltpu.SemaphoreType.DMA,
          pltpu.SemaphoreType.DMA, pltpu.SemaphoreType.DMA)),
    ],
)
def _sc_segsum(h_hbm, src_hbm, dstf_hbm, zeros_hbm, agg_hbm,
               src_v, dstr, rows_v, acc, sems):
    gsem, dsem = sems
    c = lax.axis_index("c")
    s = lax.axis_index("s")
    wid = c * _NS + s
    base_d = wid * _NCHUNK

    def gather(j, b):
        pltpu.async_copy(h_hbm.at[src_v.at[j]], rows_v.at[b], gsem[b])

    def gwait(b):
        pltpu.make_async_copy(h_hbm.at[pl.ds(0, _K)], rows_v.at[b],
                              gsem[b]).wait()

    def ifetch(j, slot):
        pltpu.async_copy(dstf_hbm.at[base_d + j], dstr.at[slot], dsem[slot])

    def dwait(slot):
        pltpu.make_async_copy(dstf_hbm.at[0], dstr.at[slot],
                              dsem[slot]).wait()

    def scat(b, slot):
        pltpu.sync_copy(rows_v.at[b], acc.at[dstr.at[slot]], add=True)

    def step(cj, slot, b, fetch_next, issue_next):
        gwait(b)
        if issue_next:
            gather(cj + 1, 1 - b)
        dwait(slot)
        scat(b, slot)
        if fetch_next:
            ifetch(cj + 4, slot)

    for m in range(_M):
        pltpu.sync_copy(src_hbm.at[m * _NW + wid], src_v)
        for slot in range(4):
            ifetch(slot, slot)
        pltpu.sync_copy(zeros_hbm, acc.at[pl.ds(s * _RPT, _RPT)])
        plsc.subcore_barrier()

        gather(0, 0)

        def quad(j4, carry):
            q = 4 * j4
            step(q, 0, 0, True, True)
            step(q + 1, 1, 1, True, True)
            step(q + 2, 2, 0, True, True)
            step(q + 3, 3, 1, True, True)
            return carry

        lax.fori_loop(0, _NCHUNK // 4 - 1, quad, 0)
        qe = _NCHUNK - 4
        step(qe, 0, 0, False, True)
        step(qe + 1, 1, 1, False, True)
        step(qe + 2, 2, 0, False, True)
        step(qe + 3, 3, 1, False, False)
        plsc.subcore_barrier()
        row0 = (2 * m + c) * _NPAD + s * _RPT
        pltpu.sync_copy(acc.at[pl.ds(s * _RPT, _RPT)],
                        agg_hbm.at[pl.ds(row0, _RPT)])


_BN = 1000
_NB = _N // _BN


def _h0_body(x_ref, wi_ref, out_ref):
    x = x_ref[...]
    for m in range(_M):
        out_ref[m] = jnp.maximum(lax.dot(x, wi_ref[m]), 0.0)


_h0_call = pl.pallas_call(
    _h0_body,
    grid=(_NB,),
    in_specs=[
        pl.BlockSpec((_BN, _D), lambda i: (i, 0)),
        pl.BlockSpec((_M, _D, _H), lambda i: (0, 0, 0)),
    ],
    out_specs=pl.BlockSpec((_M, _BN, _H), lambda i: (0, i, 0)),
    out_shape=jax.ShapeDtypeStruct((_M, _N, _H), jnp.float32),
)


def _upd_body(h0_ref, agg_ref, wh_ref, out_ref):
    for m in range(_M):
        a = agg_ref[m, 0] + agg_ref[m, 1]
        out_ref[m] = jnp.maximum(
            h0_ref[m] + lax.dot(a, wh_ref[m]), 0.0)


_upd_call = pl.pallas_call(
    _upd_body,
    grid=(_NB,),
    in_specs=[
        pl.BlockSpec((_M, _BN, _H), lambda i: (0, i, 0)),
        pl.BlockSpec((_M, 2, _BN, _H), lambda i: (0, 0, i, 0)),
        pl.BlockSpec((_M, _H, _H), lambda i: (0, 0, 0)),
    ],
    out_specs=pl.BlockSpec((_M, _BN, _H), lambda i: (0, i, 0)),
    out_shape=jax.ShapeDtypeStruct((_M, _N, _H), jnp.float32),
)


def _readout_body(x_ref, h_ref, wo_ref, bo_ref, w1_ref, b1_ref, w2_ref,
                  b2_ref, cw1_ref, cb1_ref, cw2_ref, cb2_ref, cw3_ref,
                  cb3_ref, out_ref, acc_ref):
    i = pl.program_id(0)

    @pl.when(i == 0)
    def _():
        acc_ref[...] = jnp.zeros_like(acc_ref)

    x = x_ref[...]
    for m in range(_M):
        ah = jnp.maximum(
            lax.dot(x, wo_ref[m, :_D, :])
            + lax.dot(h_ref[m], wo_ref[m, _D:, :])
            + bo_ref[m][None, :], 0.0)
        acc_ref[m, :] = acc_ref[m, :] + jnp.sum(ah, axis=0)

    @pl.when(i == _NB - 1)
    def _():
        def rb(v):
            return v.astype(jnp.bfloat16).astype(jnp.float32)

        total = 0.0
        for m in range(_M):
            e = rb(acc_ref[m, :] * (1.0 / _N))
            t = jnp.maximum(
                jnp.sum(e[:, None] * rb(w1_ref[m]), axis=0) + b1_ref[m], 0.0)
            temp = jnp.sum(rb(t) * rb(w2_ref[m])) + b2_ref[m]
            z = jnp.maximum(
                jnp.sum(e[:, None] * rb(cw1_ref[m]), axis=0) + cb1_ref[m], 0.0)
            z2 = jnp.maximum(
                jnp.sum(rb(z)[:, None] * rb(cw2_ref[m]), axis=0) + cb2_ref[m], 0.0)
            coef = jnp.sum(rb(z2) * rb(cw3_ref[m])) + cb3_ref[m]
            total = total + temp * coef
        out_ref[...] = jnp.reshape(total, (1, 1))


_readout_call = pl.pallas_call(
    _readout_body,
    grid=(_NB,),
    in_specs=[
        pl.BlockSpec((_BN, _D), lambda i: (i, 0)),
        pl.BlockSpec((_M, _BN, _H), lambda i: (0, i, 0)),
        pl.BlockSpec((_M, _D + _H, _H), lambda i: (0, 0, 0)),
        pl.BlockSpec((_M, _H), lambda i: (0, 0)),
        pl.BlockSpec((_M, _H, _H), lambda i: (0, 0, 0)),
        pl.BlockSpec((_M, _H), lambda i: (0, 0)),
        pl.BlockSpec((_M, _H), lambda i: (0, 0)),
        pl.BlockSpec((_M,), lambda i: (0,)),
        pl.BlockSpec((_M, _H, _H3), lambda i: (0, 0, 0)),
        pl.BlockSpec((_M, _H3), lambda i: (0, 0)),
        pl.BlockSpec((_M, _H3, _H9), lambda i: (0, 0, 0)),
        pl.BlockSpec((_M, _H9), lambda i: (0, 0)),
        pl.BlockSpec((_M, _H9), lambda i: (0, 0)),
        pl.BlockSpec((_M,), lambda i: (0,)),
    ],
    out_specs=pl.BlockSpec((1, 1), lambda i: (0, 0)),
    out_shape=jax.ShapeDtypeStruct((1, 1), jnp.float32),
    scratch_shapes=[pltpu.VMEM((_M, _H), jnp.float32)],
)


def kernel(x, edge_index, W_i, W_h, W_o, b_o, ffn_W1, ffn_b1, ffn_W2, ffn_b2,
           c_W1, c_b1, c_W2, c_b2, c_W3, c_b3):
    src = edge_index[0]
    dst = edge_index[1]
    npad_e = _EPTP - _EPT
    src_r = jnp.concatenate(
        [src.reshape(_NW, _EPT),
         jnp.zeros((_NW, npad_e), jnp.int32)], axis=1)
    pad_rows = (_N + (jnp.arange(_NW, dtype=jnp.int32)[:, None] * 7
                      + jnp.arange(npad_e, dtype=jnp.int32)[None, :])
                % (_NPAD - _N))
    dst_rp = jnp.concatenate([dst.reshape(_NW, _EPT), pad_rows], axis=1)
    offs = (jnp.arange(_M, dtype=jnp.int32) * _N)[:, None, None]
    src_m = (src_r[None] + offs).reshape(_M * _NW, _NCHUNK, _K)
    dst_r = dst_rp.reshape(_NW * _NCHUNK, _K)
    zeros = jnp.zeros((_RPT, _H), jnp.float32)

    h0 = _h0_call(x, W_i)
    h = h0
    for _ in range(_DEPTH):
        agg_flat = _sc_segsum(h.reshape(_M * _N, _H), src_m, dst_r, zeros)
        agg = agg_flat.reshape(_M, 2, _NPAD, _H)
        h = _upd_call(h0, agg, W_h)
    out = _readout_call(x, h, W_o, b_o, ffn_W1, ffn_b1, ffn_W2[..., 0],
                        ffn_b2[..., 0], c_W1, c_b1, c_W2, c_b2, c_W3[..., 0],
                        c_b3[..., 0])
    return out


# speedup vs baseline: 2.4663x; 1.2665x over previous
import functools

import jax
import jax.numpy as jnp
from jax import lax
from jax.experimental import pallas as pl
from jax.experimental.pallas import tpu as pltpu
from jax.experimental.pallas import tpu_sc as plsc

_N = 10000
_E = 320000
_D = 128
_H = 128
_M = 3
_DEPTH = 3
_H3 = _H // 3
_H9 = _H3 // 3

_NC = 2
_NS = 16
_NW = _NC * _NS
_EPT = _E // _NW
_K = 125
_EPTP = 10000
_NCHUNK = _EPTP // _K
_NPAD = 10240
_RPT = _NPAD // _NS

@functools.partial(
    pl.kernel,
    out_type=jax.ShapeDtypeStruct((_M * 2 * _NPAD, _H), jnp.float32),
    mesh=plsc.VectorSubcoreMesh(core_axis_name="c", subcore_axis_name="s"),
    scratch_types=[
        pltpu.VMEM((_NCHUNK, _K), jnp.int32),
        pltpu.VMEM((_NCHUNK, _K), jnp.int32),
        pltpu.VMEM((_K, _H), jnp.float32),
        pltpu.VMEM_SHARED((_NPAD, _H), jnp.float32),
        pltpu.SemaphoreType.DMA,
    ],
)
def _sc_segsum(h_hbm, src_hbm, dst_hbm, zeros_hbm, agg_hbm,
               src_v, dst_v, rows_v, acc, sem):
    c = lax.axis_index("c")
    s = lax.axis_index("s")
    wid = c * _NS + s
    pltpu.sync_copy(dst_hbm.at[wid], dst_v)
    for m in range(_M):
        pltpu.sync_copy(src_hbm.at[m * _NW + wid], src_v)
        pltpu.sync_copy(zeros_hbm, acc.at[pl.ds(s * _RPT, _RPT)])
        plsc.subcore_barrier()

        def chunk(j, carry):
            pltpu.async_copy(h_hbm.at[src_v.at[j]], rows_v, sem).wait()
            pltpu.sync_copy(rows_v, acc.at[dst_v.at[j]], add=True)
            return carry

        lax.fori_loop(0, _NCHUNK, chunk, 0)
        plsc.subcore_barrier()
        row0 = (2 * m + c) * _NPAD + s * _RPT
        pltpu.sync_copy(acc.at[pl.ds(s * _RPT, _RPT)],
                        agg_hbm.at[pl.ds(row0, _RPT)])


_BN = 1000
_NB = _N // _BN


def _h0_body(x_ref, wi_ref, out_ref):
    x = x_ref[...]
    for m in range(_M):
        out_ref[m] = jnp.maximum(lax.dot(x, wi_ref[m]), 0.0)


_h0_call = pl.pallas_call(
    _h0_body,
    grid=(_NB,),
    in_specs=[
        pl.BlockSpec((_BN, _D), lambda i: (i, 0)),
        pl.BlockSpec((_M, _D, _H), lambda i: (0, 0, 0)),
    ],
    out_specs=pl.BlockSpec((_M, _BN, _H), lambda i: (0, i, 0)),
    out_shape=jax.ShapeDtypeStruct((_M, _N, _H), jnp.float32),
)


def _upd_body(h0_ref, agg_ref, wh_ref, out_ref):
    for m in range(_M):
        a = agg_ref[m, 0] + agg_ref[m, 1]
        out_ref[m] = jnp.maximum(
            h0_ref[m] + lax.dot(a, wh_ref[m]), 0.0)


_upd_call = pl.pallas_call(
    _upd_body,
    grid=(_NB,),
    in_specs=[
        pl.BlockSpec((_M, _BN, _H), lambda i: (0, i, 0)),
        pl.BlockSpec((_M, 2, _BN, _H), lambda i: (0, 0, i, 0)),
        pl.BlockSpec((_M, _H, _H), lambda i: (0, 0, 0)),
    ],
    out_specs=pl.BlockSpec((_M, _BN, _H), lambda i: (0, i, 0)),
    out_shape=jax.ShapeDtypeStruct((_M, _N, _H), jnp.float32),
)


def _readout_body(x_ref, h_ref, wo_ref, bo_ref, w1_ref, b1_ref, w2_ref,
                  b2_ref, cw1_ref, cb1_ref, cw2_ref, cb2_ref, cw3_ref,
                  cb3_ref, out_ref, acc_ref):
    i = pl.program_id(0)

    @pl.when(i == 0)
    def _():
        acc_ref[...] = jnp.zeros_like(acc_ref)

    x = x_ref[...]
    for m in range(_M):
        ah = jnp.maximum(
            lax.dot(x, wo_ref[m, :_D, :])
            + lax.dot(h_ref[m], wo_ref[m, _D:, :])
            + bo_ref[m][None, :], 0.0)
        acc_ref[m, :] = acc_ref[m, :] + jnp.sum(ah, axis=0)

    @pl.when(i == _NB - 1)
    def _():
        def rb(v):
            return v.astype(jnp.bfloat16).astype(jnp.float32)

        total = 0.0
        for m in range(_M):
            e = rb(acc_ref[m, :] * (1.0 / _N))
            t = jnp.maximum(
                jnp.sum(e[:, None] * rb(w1_ref[m]), axis=0) + b1_ref[m], 0.0)
            temp = jnp.sum(rb(t) * rb(w2_ref[m])) + b2_ref[m]
            z = jnp.maximum(
                jnp.sum(e[:, None] * rb(cw1_ref[m]), axis=0) + cb1_ref[m], 0.0)
            z2 = jnp.maximum(
                jnp.sum(rb(z)[:, None] * rb(cw2_ref[m]), axis=0) + cb2_ref[m], 0.0)
            coef = jnp.sum(rb(z2) * rb(cw3_ref[m])) + cb3_ref[m]
            total = total + temp * coef
        out_ref[...] = jnp.reshape(total, (1, 1))


_readout_call = pl.pallas_call(
    _readout_body,
    grid=(_NB,),
    in_specs=[
        pl.BlockSpec((_BN, _D), lambda i: (i, 0)),
        pl.BlockSpec((_M, _BN, _H), lambda i: (0, i, 0)),
        pl.BlockSpec((_M, _D + _H, _H), lambda i: (0, 0, 0)),
        pl.BlockSpec((_M, _H), lambda i: (0, 0)),
        pl.BlockSpec((_M, _H, _H), lambda i: (0, 0, 0)),
        pl.BlockSpec((_M, _H), lambda i: (0, 0)),
        pl.BlockSpec((_M, _H), lambda i: (0, 0)),
        pl.BlockSpec((_M,), lambda i: (0,)),
        pl.BlockSpec((_M, _H, _H3), lambda i: (0, 0, 0)),
        pl.BlockSpec((_M, _H3), lambda i: (0, 0)),
        pl.BlockSpec((_M, _H3, _H9), lambda i: (0, 0, 0)),
        pl.BlockSpec((_M, _H9), lambda i: (0, 0)),
        pl.BlockSpec((_M, _H9), lambda i: (0, 0)),
        pl.BlockSpec((_M,), lambda i: (0,)),
    ],
    out_specs=pl.BlockSpec((1, 1), lambda i: (0, 0)),
    out_shape=jax.ShapeDtypeStruct((1, 1), jnp.float32),
    scratch_shapes=[pltpu.VMEM((_M, _H), jnp.float32)],
)


def kernel(x, edge_index, W_i, W_h, W_o, b_o, ffn_W1, ffn_b1, ffn_W2, ffn_b2,
           c_W1, c_b1, c_W2, c_b2, c_W3, c_b3):
    src = edge_index[0]
    dst = edge_index[1]
    offs = (jnp.arange(_M, dtype=jnp.int32) * _N)[:, None]
    src_m = (src[None, :] + offs).reshape(_M * _NW, _NCHUNK, _K)
    dst_r = dst.reshape(_NW, _NCHUNK, _K)
    zeros = jnp.zeros((_RPT, _H), jnp.float32)

    h0 = _h0_call(x, W_i)
    h = h0
    for _ in range(_DEPTH):
        agg_flat = _sc_segsum(h.reshape(_M * _N, _H), src_m, dst_r, zeros)
        agg = agg_flat.reshape(_M, 2, _NPAD, _H)
        h = _upd_call(h0, agg, W_h)
    out = _readout_call(x, h, W_o, b_o, ffn_W1, ffn_b1, ffn_W2[..., 0],
                        ffn_b2[..., 0], c_W1, c_b1, c_W2, c_b2, c_W3[..., 0],
                        c_b3[..., 0])
    return out
